# baseline (speedup 1.0000x reference)
import functools

import jax
import jax.numpy as jnp
import numpy as np
from jax import lax
from jax.experimental import pallas as pl
from jax.experimental.pallas import tpu as pltpu
from jax.experimental.pallas import tpu_sc as plsc

N_NODES = 10000
N_PAD = 10240
D_IN = 128
C_OUT = 64
C_HALF = 32
E_EDGES = 320000
K_HOPS = 16
ALPHA = 0.05
COEF = (1.0 - ALPHA) / K_HOPS

EC = 128
EROWS = 2560
EROWS_PER_TILE = EROWS // 16
EROWS_PER_TILE32 = EROWS // 32
NPT = N_PAD // 16
NCH = NPT // 128

_F = np.array([32 * (t // 32) + (2 * (t % 32) if t % 32 < 16
                                 else 2 * ((t % 32) - 16) + 1)
               for t in range(C_OUT)])
_F_INV = np.argsort(_F)

_mesh = plsc.VectorSubcoreMesh(
    core_axis_name="c", subcore_axis_name="s", num_cores=2, num_subcores=16)

_INTER = plsc.PackFormat.INTERLEAVED


def _fill(ref, n16, val, dtype):
  def body(i, _):
    ref[pl.ds(i * 16, 16)] = jnp.full((16,), val, dtype)
    return 0
  lax.fori_loop(0, n16, body, 0)


def _fill2d(ref, rows, cols, val, dtype):
  def body(i, _):
    r = i // (cols // 16)
    g = i % (cols // 16)
    ref[r, pl.ds(g * 16, 16)] = jnp.full((16,), val, dtype)
    return 0
  lax.fori_loop(0, rows * (cols // 16), body, 0)


@functools.partial(
    pl.kernel,
    out_type=jax.ShapeDtypeStruct((2, N_PAD), jnp.float32),
    mesh=_mesh,
    scratch_types=[
        pltpu.VMEM((EROWS_PER_TILE32, EC), jnp.int32),
        pltpu.VMEM((EC,), jnp.float32),
        pltpu.VMEM((NPT,), jnp.float32),
        pltpu.VMEM_SHARED((N_PAD,), jnp.float32),
    ],
)
def _sc_degree(col_hbm, deg_out, colb, ones, zb, degsh):
  cid = lax.axis_index("c")
  sid = lax.axis_index("s")
  tid = sid * 2 + cid
  pltpu.sync_copy(col_hbm.at[pl.ds(tid * EROWS_PER_TILE32, EROWS_PER_TILE32)],
                  colb)
  _fill(ones, EC // 16, 1.0, jnp.float32)
  _fill(zb, NPT // 16, 0.0, jnp.float32)
  pltpu.sync_copy(zb, degsh.at[pl.ds(sid * NPT, NPT)])
  plsc.subcore_barrier()

  def chunk(j, _):
    pltpu.sync_copy(ones, degsh.at[colb.at[j]], add=True)
    return 0
  lax.fori_loop(0, EROWS_PER_TILE32, chunk, 0)
  plsc.subcore_barrier()
  pltpu.sync_copy(degsh.at[pl.ds(sid * NPT, NPT)],
                  deg_out.at[cid, pl.ds(sid * NPT, NPT)])


def _tc_prep_body(x_ref, wcat_ref, y0_ref, yab_ref):
  yy = jnp.dot(x_ref[...], wcat_ref[...], preferred_element_type=jnp.float32)
  y0_ref[...] = yy[:, :C_OUT]
  yab_ref[0] = yy[:, C_OUT:C_OUT + C_HALF].astype(jnp.bfloat16)
  yab_ref[1] = yy[:, C_OUT + C_HALF:].astype(jnp.bfloat16)


def _tc_prep(x_pad, wcat):
  blk = 512
  grid = (N_PAD // blk,)
  return pl.pallas_call(
      _tc_prep_body,
      grid=grid,
      in_specs=[
          pl.BlockSpec((blk, D_IN), lambda i: (i, 0)),
          pl.BlockSpec((D_IN, 2 * C_OUT), lambda i: (0, 0)),
      ],
      out_specs=[
          pl.BlockSpec((blk, C_OUT), lambda i: (i, 0)),
          pl.BlockSpec((2, blk, C_HALF), lambda i: (0, i, 0)),
      ],
      out_shape=[
          jax.ShapeDtypeStruct((N_PAD, C_OUT), jnp.float32),
          jax.ShapeDtypeStruct((2, N_PAD, C_HALF), jnp.bfloat16),
      ],
  )(x_pad, wcat)


@functools.partial(
    pl.kernel,
    out_type=jax.ShapeDtypeStruct((2, N_PAD, C_HALF), jnp.float32),
    mesh=_mesh,
    scratch_types=[
        pltpu.VMEM((EROWS_PER_TILE + 8, EC), jnp.int32),
        pltpu.VMEM((EROWS_PER_TILE, EC), jnp.int32),
        pltpu.VMEM((EC, C_HALF), jnp.bfloat16),
        pltpu.VMEM((EC, C_HALF), jnp.bfloat16),
        pltpu.VMEM((EC, C_HALF), jnp.bfloat16),
        pltpu.VMEM((EC, C_HALF), jnp.bfloat16),
        pltpu.VMEM((EC, C_HALF), jnp.bfloat16),
        pltpu.VMEM((NPT, C_HALF), jnp.float32),
        pltpu.VMEM((NPT,), jnp.float32),
        pltpu.VMEM((NPT,), jnp.float32),
        pltpu.VMEM((NPT,), jnp.float32),
        pltpu.VMEM((NPT,), jnp.float32),
        pltpu.VMEM((NPT,), jnp.float32),
        pltpu.VMEM_SHARED((N_PAD, C_HALF), jnp.bfloat16),
        pltpu.VMEM_SHARED((N_PAD, C_HALF), jnp.bfloat16),
        pltpu.SemaphoreType.DMA,
        pltpu.SemaphoreType.DMA,
        pltpu.SemaphoreType.DMA,
        pltpu.SemaphoreType.DMA,
    ],
    compiler_params=pltpu.CompilerParams(
        use_tc_tiling_on_sc=False, needs_layout_passes=False),
)
def _sc_hops(yab_hbm, row_hbm, col_hbm, degp_hbm, s_out,
             rowb, colb, eb0, eb1, eb2, eb3, zbuf, sbuf,
             dp0, dp1, d1b, dnb, dsqb, u_sh, acc_sh,
             sem_g, sem_s, sem_w, sem_z):
  abuf, ubuf = eb2, eb3
  cid = lax.axis_index("c")
  sid = lax.axis_index("s")
  nbase = sid * NPT

  pltpu.sync_copy(row_hbm.at[pl.ds(sid * EROWS_PER_TILE, EROWS_PER_TILE)],
                  rowb.at[pl.ds(0, EROWS_PER_TILE)])
  def padrow(i, _):
    rowb[EROWS_PER_TILE + i // 8, pl.ds((i % 8) * 16, 16)] = jnp.full(
        (16,), N_NODES, jnp.int32)
    return 0
  lax.fori_loop(0, 8 * 8, padrow, 0)
  pltpu.sync_copy(col_hbm.at[pl.ds(sid * EROWS_PER_TILE, EROWS_PER_TILE)], colb)

  pltpu.sync_copy(degp_hbm.at[0, pl.ds(nbase, NPT)], dp0)
  pltpu.sync_copy(degp_hbm.at[1, pl.ds(nbase, NPT)], dp1)

  def dblk(t, _):
    sl = pl.ds(t * 16, 16)
    deg = dp0[sl] + dp1[sl] + 1.0
    i = plsc.bitcast(deg, jnp.int32)
    y = plsc.bitcast(jnp.int32(0x5F3759DF) - (i >> 1), jnp.float32)
    y = y * (1.5 - 0.5 * deg * y * y)
    y = y * (1.5 - 0.5 * deg * y * y)
    y = y * (1.5 - 0.5 * deg * y * y)
    dnb[sl] = y
    d1b[sl] = y * y
    dsqb[sl] = deg * y
    return 0
  lax.fori_loop(0, NPT // 16, dblk, 0)

  def zrow(i, _):
    zbuf[i, :] = jnp.zeros((C_HALF,), jnp.bfloat16)
    return 0
  lax.fori_loop(0, EC, zrow, 0)
  _fill2d(sbuf, NPT, C_HALF, 0.0, jnp.float32)

  def uchunk(j, _):
    base = nbase + j * EC
    pltpu.sync_copy(yab_hbm.at[cid, pl.ds(base, EC)], ubuf)

    def rblk(t, _):
      dvec = dnb[pl.ds(j * EC + t * 16, 16)]
      for rr in range(16):
        r = t * 16 + rr
        b0, b1 = plsc.unpack(ubuf[r, :], format=_INTER)
        ubuf[r, :] = plsc.pack(b0 * dvec[rr], b1 * dvec[rr], format=_INTER)
      return 0
    lax.fori_loop(0, EC // 16, rblk, 0)
    pltpu.sync_copy(ubuf, u_sh.at[pl.ds(base, EC)])
    pltpu.sync_copy(zbuf, acc_sh.at[pl.ds(base, EC)])
    return 0
  lax.fori_loop(0, NCH, uchunk, 0)
  plsc.subcore_barrier()

  def k_body(k, _):
    def wait_g(buf):
      pltpu.make_async_copy(u_sh.at[rowb.at[0]], buf, sem_g).wait()

    def wait_s(buf):
      pltpu.make_async_copy(buf, acc_sh.at[colb.at[0]], sem_s).wait()

    def gath(j, buf):
      pltpu.async_copy(u_sh.at[rowb.at[j]], buf, sem_g)

    def scat(j, buf):
      pltpu.async_copy(buf, acc_sh.at[colb.at[j]], sem_s, add=True)

    gath(0, eb0)
    gath(1, eb1)

    def estep(jj, _):
      j0 = 4 * jj
      wait_g(eb0)
      gath(j0 + 2, eb2)
      scat(j0, eb0)
      wait_g(eb1)
      gath(j0 + 3, eb3)
      scat(j0 + 1, eb1)
      wait_s(eb0)
      gath(j0 + 4, eb0)
      wait_g(eb2)
      scat(j0 + 2, eb2)
      wait_s(eb1)
      gath(j0 + 5, eb1)
      wait_g(eb3)
      scat(j0 + 3, eb3)
      wait_s(eb2)
      wait_s(eb3)
      return 0
    lax.fori_loop(0, EROWS_PER_TILE // 4, estep, 0)
    wait_g(eb0)
    wait_g(eb1)
    plsc.subcore_barrier()

    def nsl(j):
      return pl.ds(nbase + j * EC, EC)

    def rblk_fn(j, a, ub):
      def rblk(t, _):
        d1vec = d1b[pl.ds(j * EC + t * 16, 16)]
        for rr in range(16):
          r = t * 16 + rr
          dscale = d1vec[rr]
          a0, a1 = plsc.unpack(a[r, :], format=_INTER)
          b0, b1 = plsc.unpack(ub[r, :], format=_INTER)
          v0 = (a0 + b0) * dscale
          v1 = (a1 + b1) * dscale
          ub[r, :] = plsc.pack(v0, v1, format=_INTER)
          sl0 = pl.ds(0, 16)
          sl1 = pl.ds(16, 16)
          sbuf[j * EC + r, sl0] = sbuf[j * EC + r, sl0] + v0
          sbuf[j * EC + r, sl1] = sbuf[j * EC + r, sl1] + v1
        return 0
      lax.fori_loop(0, EC // 16, rblk, 0)

    bufs = [(eb0, eb1), (eb2, eb3)]
    pltpu.async_copy(acc_sh.at[nsl(0)], eb0, sem_g)
    pltpu.async_copy(u_sh.at[nsl(0)], eb1, sem_g)
    for j in range(NCH):
      a, ub = bufs[j % 2]
      pltpu.make_async_copy(acc_sh.at[nsl(0)], a, sem_g).wait()
      pltpu.make_async_copy(u_sh.at[nsl(0)], ub, sem_g).wait()
      if j >= 1:
        pltpu.make_async_copy(ub, u_sh.at[nsl(0)], sem_w).wait()
      if j + 1 < NCH:
        na, nb = bufs[(j + 1) % 2]
        pltpu.async_copy(acc_sh.at[nsl(j + 1)], na, sem_g)
        pltpu.async_copy(u_sh.at[nsl(j + 1)], nb, sem_g)
      pltpu.async_copy(zbuf, acc_sh.at[nsl(j)], sem_z)
      rblk_fn(j, a, ub)
      pltpu.async_copy(ub, u_sh.at[nsl(j)], sem_w)
    pltpu.make_async_copy(eb1, u_sh.at[nsl(0)], sem_w).wait()
    for j in range(NCH):
      pltpu.make_async_copy(zbuf, acc_sh.at[nsl(0)], sem_z).wait()
    plsc.subcore_barrier()
    return 0
  lax.fori_loop(0, K_HOPS, k_body, 0)

  def oblk(t, _):
    dvec = dsqb[pl.ds(t * 16, 16)]
    for rr in range(16):
      r = t * 16 + rr
      for g in range(2):
        sl = pl.ds(g * 16, 16)
        sbuf[r, sl] = sbuf[r, sl] * dvec[rr]
    return 0
  lax.fori_loop(0, NPT // 16, oblk, 0)
  pltpu.sync_copy(sbuf, s_out.at[cid, pl.ds(nbase, NPT)])


def _tc_final_body(y0_ref, s2_ref, b_ref, p_ref, o_ref):
  s_cat = jnp.concatenate([s2_ref[0], s2_ref[1]], axis=1)
  logits = ALPHA * y0_ref[...] + COEF * s_cat + b_ref[...]
  m = jnp.max(logits, axis=1, keepdims=True)
  ex = jnp.exp(logits - m)
  lse = jnp.log(jnp.sum(ex, axis=1, keepdims=True)) + m
  o_ref[...] = jnp.dot(logits - lse, p_ref[...],
                       preferred_element_type=jnp.float32,
                       precision=lax.Precision.HIGHEST)


def _tc_final(y0, s2, b2, pmat):
  blk = 512
  grid = (N_PAD // blk,)
  return pl.pallas_call(
      _tc_final_body,
      grid=grid,
      in_specs=[
          pl.BlockSpec((blk, C_OUT), lambda i: (i, 0)),
          pl.BlockSpec((2, blk, C_HALF), lambda i: (0, i, 0)),
          pl.BlockSpec((1, C_OUT), lambda i: (0, 0)),
          pl.BlockSpec((C_OUT, C_OUT), lambda i: (0, 0)),
      ],
      out_specs=pl.BlockSpec((blk, C_OUT), lambda i: (i, 0)),
      out_shape=jax.ShapeDtypeStruct((N_PAD, C_OUT), jnp.float32),
  )(y0, s2, b2, pmat)


def kernel(x, edge_index, W, b):
  row = edge_index[0]
  col = edge_index[1]
  pad = EROWS * EC - E_EDGES
  rowp = jnp.concatenate(
      [row, jnp.full((pad,), N_NODES, jnp.int32)]).reshape(EROWS, EC)
  colp = jnp.concatenate(
      [col, jnp.full((pad,), N_NODES, jnp.int32)]).reshape(EROWS, EC)
  x_pad = jnp.pad(x, ((0, N_PAD - N_NODES), (0, 0)))
  wcat = jnp.concatenate([W[:, _F], W], axis=1)

  y0, yab = _tc_prep(x_pad, wcat)
  degp = _sc_degree(colp)
  s2 = _sc_hops(yab, rowp, colp, degp)
  pmat = jnp.asarray(np.eye(C_OUT, dtype=np.float32)[:, _F].T)
  out = _tc_final(y0, s2, b[_F].reshape(1, C_OUT), pmat)
  return out[:N_NODES]


# speedup vs baseline: 1.0851x; 1.0851x over previous
import functools

import jax
import jax.numpy as jnp
import numpy as np
from jax import lax
from jax.experimental import pallas as pl
from jax.experimental.pallas import tpu as pltpu
from jax.experimental.pallas import tpu_sc as plsc

N_NODES = 10000
N_PAD = 10240
D_IN = 128
C_OUT = 64
C_HALF = 32
E_EDGES = 320000
K_HOPS = 16
ALPHA = 0.05
COEF = (1.0 - ALPHA) / K_HOPS

EC = 128
EROWS = 2560
EROWS_PER_TILE = EROWS // 16
EROWS_PER_TILE32 = EROWS // 32
NPT = N_PAD // 16
NCH = NPT // 128

_F = np.array([32 * (t // 32) + (2 * (t % 32) if t % 32 < 16
                                 else 2 * ((t % 32) - 16) + 1)
               for t in range(C_OUT)])
_F_INV = np.argsort(_F)

_mesh = plsc.VectorSubcoreMesh(
    core_axis_name="c", subcore_axis_name="s", num_cores=2, num_subcores=16)

_INTER = plsc.PackFormat.INTERLEAVED


def _fill(ref, n16, val, dtype):
  def body(i, _):
    ref[pl.ds(i * 16, 16)] = jnp.full((16,), val, dtype)
    return 0
  lax.fori_loop(0, n16, body, 0)


def _fill2d(ref, rows, cols, val, dtype):
  def body(i, _):
    r = i // (cols // 16)
    g = i % (cols // 16)
    ref[r, pl.ds(g * 16, 16)] = jnp.full((16,), val, dtype)
    return 0
  lax.fori_loop(0, rows * (cols // 16), body, 0)


@functools.partial(
    pl.kernel,
    out_type=jax.ShapeDtypeStruct((2, N_PAD), jnp.float32),
    mesh=_mesh,
    scratch_types=[
        pltpu.VMEM((EROWS_PER_TILE32, EC), jnp.int32),
        pltpu.VMEM((EC,), jnp.float32),
        pltpu.VMEM((NPT,), jnp.float32),
        pltpu.VMEM_SHARED((N_PAD,), jnp.float32),
    ],
)
def _sc_degree(col_hbm, deg_out, colb, ones, zb, degsh):
  cid = lax.axis_index("c")
  sid = lax.axis_index("s")
  tid = sid * 2 + cid
  pltpu.sync_copy(col_hbm.at[pl.ds(tid * EROWS_PER_TILE32, EROWS_PER_TILE32)],
                  colb)
  _fill(ones, EC // 16, 1.0, jnp.float32)
  _fill(zb, NPT // 16, 0.0, jnp.float32)
  pltpu.sync_copy(zb, degsh.at[pl.ds(sid * NPT, NPT)])
  plsc.subcore_barrier()

  def chunk(j, _):
    pltpu.sync_copy(ones, degsh.at[colb.at[j]], add=True)
    return 0
  lax.fori_loop(0, EROWS_PER_TILE32, chunk, 0)
  plsc.subcore_barrier()
  pltpu.sync_copy(degsh.at[pl.ds(sid * NPT, NPT)],
                  deg_out.at[cid, pl.ds(sid * NPT, NPT)])


def _tc_prep_body(x_ref, wcat_ref, y0_ref, yab_ref):
  yy = jnp.dot(x_ref[...], wcat_ref[...], preferred_element_type=jnp.float32)
  y0_ref[...] = yy[:, :C_OUT]
  yab_ref[0] = yy[:, C_OUT:C_OUT + C_HALF].astype(jnp.bfloat16)
  yab_ref[1] = yy[:, C_OUT + C_HALF:].astype(jnp.bfloat16)


def _tc_prep(x_pad, wcat):
  blk = 512
  grid = (N_PAD // blk,)
  return pl.pallas_call(
      _tc_prep_body,
      grid=grid,
      in_specs=[
          pl.BlockSpec((blk, D_IN), lambda i: (i, 0)),
          pl.BlockSpec((D_IN, 2 * C_OUT), lambda i: (0, 0)),
      ],
      out_specs=[
          pl.BlockSpec((blk, C_OUT), lambda i: (i, 0)),
          pl.BlockSpec((2, blk, C_HALF), lambda i: (0, i, 0)),
      ],
      out_shape=[
          jax.ShapeDtypeStruct((N_PAD, C_OUT), jnp.float32),
          jax.ShapeDtypeStruct((2, N_PAD, C_HALF), jnp.bfloat16),
      ],
  )(x_pad, wcat)


@functools.partial(
    pl.kernel,
    out_type=jax.ShapeDtypeStruct((2, N_PAD, C_HALF), jnp.float32),
    mesh=_mesh,
    scratch_types=[
        pltpu.VMEM((EROWS_PER_TILE + 8, EC), jnp.int32),
        pltpu.VMEM((EROWS_PER_TILE, EC), jnp.int32),
        pltpu.VMEM((EC, C_HALF), jnp.bfloat16),
        pltpu.VMEM((EC, C_HALF), jnp.bfloat16),
        pltpu.VMEM((EC, C_HALF), jnp.bfloat16),
        pltpu.VMEM((EC, C_HALF), jnp.bfloat16),
        pltpu.VMEM((EC, C_HALF), jnp.bfloat16),
        pltpu.VMEM((NPT, C_HALF), jnp.float32),
        pltpu.VMEM((NPT,), jnp.float32),
        pltpu.VMEM((NPT,), jnp.float32),
        pltpu.VMEM((NPT,), jnp.float32),
        pltpu.VMEM((NPT,), jnp.float32),
        pltpu.VMEM((NPT,), jnp.float32),
        pltpu.VMEM_SHARED((N_PAD, C_HALF), jnp.bfloat16),
        pltpu.VMEM_SHARED((N_PAD, C_HALF), jnp.bfloat16),
        pltpu.SemaphoreType.DMA,
        pltpu.SemaphoreType.DMA,
        pltpu.SemaphoreType.DMA,
        pltpu.SemaphoreType.DMA,
    ],
    compiler_params=pltpu.CompilerParams(
        use_tc_tiling_on_sc=False, needs_layout_passes=False),
)
def _sc_hops(yab_hbm, row_hbm, col_hbm, degp_hbm, s_out,
             rowb, colb, eb0, eb1, eb2, eb3, zbuf, sbuf,
             dp0, dp1, d1b, dnb, dsqb, u_sh, acc_sh,
             sem_g, sem_s, sem_w, sem_z):
  abuf, ubuf = eb2, eb3
  cid = lax.axis_index("c")
  sid = lax.axis_index("s")
  nbase = sid * NPT

  pltpu.sync_copy(row_hbm.at[pl.ds(sid * EROWS_PER_TILE, EROWS_PER_TILE)],
                  rowb.at[pl.ds(0, EROWS_PER_TILE)])
  def padrow(i, _):
    g = i % 8
    rowb[EROWS_PER_TILE + i // 8, pl.ds(g * 16, 16)] = (
        N_NODES + g * 16 + lax.iota(jnp.int32, 16))
    return 0
  lax.fori_loop(0, 8 * 8, padrow, 0)
  pltpu.sync_copy(col_hbm.at[pl.ds(sid * EROWS_PER_TILE, EROWS_PER_TILE)], colb)

  pltpu.sync_copy(degp_hbm.at[0, pl.ds(nbase, NPT)], dp0)
  pltpu.sync_copy(degp_hbm.at[1, pl.ds(nbase, NPT)], dp1)

  def dblk(t, _):
    sl = pl.ds(t * 16, 16)
    deg = dp0[sl] + dp1[sl] + 1.0
    i = plsc.bitcast(deg, jnp.int32)
    y = plsc.bitcast(jnp.int32(0x5F3759DF) - (i >> 1), jnp.float32)
    y = y * (1.5 - 0.5 * deg * y * y)
    y = y * (1.5 - 0.5 * deg * y * y)
    y = y * (1.5 - 0.5 * deg * y * y)
    dnb[sl] = y
    d1b[sl] = y * y
    dsqb[sl] = deg * y
    return 0
  lax.fori_loop(0, NPT // 16, dblk, 0)

  def zrow(i, _):
    zbuf[i, :] = jnp.zeros((C_HALF,), jnp.bfloat16)
    return 0
  lax.fori_loop(0, EC, zrow, 0)
  _fill2d(sbuf, NPT, C_HALF, 0.0, jnp.float32)

  def uchunk(j, _):
    base = nbase + j * EC
    pltpu.sync_copy(yab_hbm.at[cid, pl.ds(base, EC)], ubuf)

    def rblk(t, _):
      dvec = dnb[pl.ds(j * EC + t * 16, 16)]
      for rr in range(16):
        r = t * 16 + rr
        b0, b1 = plsc.unpack(ubuf[r, :], format=_INTER)
        ubuf[r, :] = plsc.pack(b0 * dvec[rr], b1 * dvec[rr], format=_INTER)
      return 0
    lax.fori_loop(0, EC // 16, rblk, 0)
    pltpu.sync_copy(ubuf, u_sh.at[pl.ds(base, EC)])
    pltpu.sync_copy(zbuf, acc_sh.at[pl.ds(base, EC)])
    return 0
  lax.fori_loop(0, NCH, uchunk, 0)
  plsc.subcore_barrier()

  def k_body(k, _):
    def wait_g(buf):
      pltpu.make_async_copy(u_sh.at[rowb.at[0]], buf, sem_g).wait()

    def wait_s(buf):
      pltpu.make_async_copy(buf, acc_sh.at[colb.at[0]], sem_s).wait()

    def gath(j, buf):
      pltpu.async_copy(u_sh.at[rowb.at[j]], buf, sem_g)

    def scat(j, buf):
      pltpu.async_copy(buf, acc_sh.at[colb.at[j]], sem_s, add=True)

    gath(0, eb0)
    gath(1, eb1)
    for _ in range(2):
      pltpu.async_copy(zbuf, acc_sh.at[rowb.at[EROWS_PER_TILE]], sem_s,
                       add=True)

    def estep(jj, _):
      j0 = 4 * jj
      wait_g(eb0)
      wait_s(eb2)
      gath(j0 + 2, eb2)
      scat(j0, eb0)
      wait_g(eb1)
      wait_s(eb3)
      gath(j0 + 3, eb3)
      scat(j0 + 1, eb1)
      wait_s(eb0)
      gath(j0 + 4, eb0)
      wait_g(eb2)
      scat(j0 + 2, eb2)
      wait_s(eb1)
      gath(j0 + 5, eb1)
      wait_g(eb3)
      scat(j0 + 3, eb3)
      return 0
    lax.fori_loop(0, EROWS_PER_TILE // 4, estep, 0)
    wait_g(eb0)
    wait_g(eb1)
    wait_s(eb2)
    wait_s(eb3)
    plsc.subcore_barrier()

    def nsl(j):
      return pl.ds(nbase + j * EC, EC)

    def rblk_fn(j, a, ub):
      def rblk(t, _):
        d1vec = d1b[pl.ds(j * EC + t * 16, 16)]
        for rr in range(16):
          r = t * 16 + rr
          dscale = d1vec[rr]
          a0, a1 = plsc.unpack(a[r, :], format=_INTER)
          b0, b1 = plsc.unpack(ub[r, :], format=_INTER)
          v0 = (a0 + b0) * dscale
          v1 = (a1 + b1) * dscale
          ub[r, :] = plsc.pack(v0, v1, format=_INTER)
          sl0 = pl.ds(0, 16)
          sl1 = pl.ds(16, 16)
          sbuf[j * EC + r, sl0] = sbuf[j * EC + r, sl0] + v0
          sbuf[j * EC + r, sl1] = sbuf[j * EC + r, sl1] + v1
        return 0
      lax.fori_loop(0, EC // 16, rblk, 0)

    bufs = [(eb0, eb1), (eb2, eb3)]
    pltpu.async_copy(acc_sh.at[nsl(0)], eb0, sem_g)
    pltpu.async_copy(u_sh.at[nsl(0)], eb1, sem_g)
    for j in range(NCH):
      a, ub = bufs[j % 2]
      pltpu.make_async_copy(acc_sh.at[nsl(0)], a, sem_g).wait()
      pltpu.make_async_copy(u_sh.at[nsl(0)], ub, sem_g).wait()
      if j >= 1:
        pltpu.make_async_copy(ub, u_sh.at[nsl(0)], sem_w).wait()
      if j + 1 < NCH:
        na, nb = bufs[(j + 1) % 2]
        pltpu.async_copy(acc_sh.at[nsl(j + 1)], na, sem_g)
        pltpu.async_copy(u_sh.at[nsl(j + 1)], nb, sem_g)
      pltpu.async_copy(zbuf, acc_sh.at[nsl(j)], sem_z)
      rblk_fn(j, a, ub)
      pltpu.async_copy(ub, u_sh.at[nsl(j)], sem_w)
    pltpu.make_async_copy(eb1, u_sh.at[nsl(0)], sem_w).wait()
    for j in range(NCH):
      pltpu.make_async_copy(zbuf, acc_sh.at[nsl(0)], sem_z).wait()
    plsc.subcore_barrier()
    return 0
  lax.fori_loop(0, K_HOPS, k_body, 0)

  def oblk(t, _):
    dvec = dsqb[pl.ds(t * 16, 16)]
    for rr in range(16):
      r = t * 16 + rr
      for g in range(2):
        sl = pl.ds(g * 16, 16)
        sbuf[r, sl] = sbuf[r, sl] * dvec[rr]
    return 0
  lax.fori_loop(0, NPT // 16, oblk, 0)
  pltpu.sync_copy(sbuf, s_out.at[cid, pl.ds(nbase, NPT)])


def _tc_final_body(y0_ref, s2_ref, b_ref, p_ref, o_ref):
  s_cat = jnp.concatenate([s2_ref[0], s2_ref[1]], axis=1)
  logits = ALPHA * y0_ref[...] + COEF * s_cat + b_ref[...]
  m = jnp.max(logits, axis=1, keepdims=True)
  ex = jnp.exp(logits - m)
  lse = jnp.log(jnp.sum(ex, axis=1, keepdims=True)) + m
  o_ref[...] = jnp.dot(logits - lse, p_ref[...],
                       preferred_element_type=jnp.float32,
                       precision=lax.Precision.HIGHEST)


def _tc_final(y0, s2, b2, pmat):
  blk = 512
  grid = (N_PAD // blk,)
  return pl.pallas_call(
      _tc_final_body,
      grid=grid,
      in_specs=[
          pl.BlockSpec((blk, C_OUT), lambda i: (i, 0)),
          pl.BlockSpec((2, blk, C_HALF), lambda i: (0, i, 0)),
          pl.BlockSpec((1, C_OUT), lambda i: (0, 0)),
          pl.BlockSpec((C_OUT, C_OUT), lambda i: (0, 0)),
      ],
      out_specs=pl.BlockSpec((blk, C_OUT), lambda i: (i, 0)),
      out_shape=jax.ShapeDtypeStruct((N_PAD, C_OUT), jnp.float32),
  )(y0, s2, b2, pmat)


def kernel(x, edge_index, W, b):
  row = edge_index[0]
  col = edge_index[1]
  pad = EROWS * EC - E_EDGES
  rowp = jnp.concatenate(
      [row, jnp.full((pad,), N_NODES, jnp.int32)]).reshape(EROWS, EC)
  colp = jnp.concatenate(
      [col, jnp.full((pad,), N_NODES, jnp.int32)]).reshape(EROWS, EC)
  x_pad = jnp.pad(x, ((0, N_PAD - N_NODES), (0, 0)))
  wcat = jnp.concatenate([W[:, _F], W], axis=1)

  y0, yab = _tc_prep(x_pad, wcat)
  degp = _sc_degree(colp)
  s2 = _sc_hops(yab, rowp, colp, degp)
  pmat = jnp.asarray(np.eye(C_OUT, dtype=np.float32)[:, _F].T)
  out = _tc_final(y0, s2, b[_F].reshape(1, C_OUT), pmat)
  return out[:N_NODES]


# speedup vs baseline: 1.0950x; 1.0092x over previous
import functools

import jax
import jax.numpy as jnp
import numpy as np
from jax import lax
from jax.experimental import pallas as pl
from jax.experimental.pallas import tpu as pltpu
from jax.experimental.pallas import tpu_sc as plsc

N_NODES = 10000
N_PAD = 10240
D_IN = 128
C_OUT = 64
C_HALF = 32
E_EDGES = 320000
K_HOPS = 16
ALPHA = 0.05
COEF = (1.0 - ALPHA) / K_HOPS

EC = 128
EROWS = 2560
EROWS_PER_TILE = EROWS // 16
EROWS_PER_TILE32 = EROWS // 32
NPT = N_PAD // 16
NCH = NPT // 128

_F = np.array([32 * (t // 32) + (2 * (t % 32) if t % 32 < 16
                                 else 2 * ((t % 32) - 16) + 1)
               for t in range(C_OUT)])
_F_INV = np.argsort(_F)

_mesh = plsc.VectorSubcoreMesh(
    core_axis_name="c", subcore_axis_name="s", num_cores=2, num_subcores=16)

_INTER = plsc.PackFormat.INTERLEAVED


def _fill(ref, n16, val, dtype):
  def body(i, _):
    ref[pl.ds(i * 16, 16)] = jnp.full((16,), val, dtype)
    return 0
  lax.fori_loop(0, n16, body, 0)


def _fill2d(ref, rows, cols, val, dtype):
  def body(i, _):
    r = i // (cols // 16)
    g = i % (cols // 16)
    ref[r, pl.ds(g * 16, 16)] = jnp.full((16,), val, dtype)
    return 0
  lax.fori_loop(0, rows * (cols // 16), body, 0)


@functools.partial(
    pl.kernel,
    out_type=jax.ShapeDtypeStruct((2, N_PAD), jnp.float32),
    mesh=_mesh,
    scratch_types=[
        pltpu.VMEM((EROWS_PER_TILE32, EC), jnp.int32),
        pltpu.VMEM((EC,), jnp.float32),
        pltpu.VMEM((NPT,), jnp.float32),
        pltpu.VMEM_SHARED((N_PAD,), jnp.float32),
    ],
)
def _sc_degree(col_hbm, deg_out, colb, ones, zb, degsh):
  cid = lax.axis_index("c")
  sid = lax.axis_index("s")
  tid = sid * 2 + cid
  pltpu.sync_copy(col_hbm.at[pl.ds(tid * EROWS_PER_TILE32, EROWS_PER_TILE32)],
                  colb)
  _fill(ones, EC // 16, 1.0, jnp.float32)
  _fill(zb, NPT // 16, 0.0, jnp.float32)
  pltpu.sync_copy(zb, degsh.at[pl.ds(sid * NPT, NPT)])
  plsc.subcore_barrier()

  def chunk(j, _):
    pltpu.sync_copy(ones, degsh.at[colb.at[j]], add=True)
    return 0
  lax.fori_loop(0, EROWS_PER_TILE32, chunk, 0)
  plsc.subcore_barrier()
  pltpu.sync_copy(degsh.at[pl.ds(sid * NPT, NPT)],
                  deg_out.at[cid, pl.ds(sid * NPT, NPT)])


def _tc_prep_body(x_ref, wcat_ref, y0_ref, yab_ref):
  yy = jnp.dot(x_ref[...], wcat_ref[...], preferred_element_type=jnp.float32)
  y0_ref[...] = yy[:, :C_OUT]
  yab_ref[0] = yy[:, C_OUT:C_OUT + C_HALF].astype(jnp.bfloat16)
  yab_ref[1] = yy[:, C_OUT + C_HALF:].astype(jnp.bfloat16)


def _tc_prep(x_pad, wcat):
  blk = 512
  grid = (N_PAD // blk,)
  return pl.pallas_call(
      _tc_prep_body,
      grid=grid,
      in_specs=[
          pl.BlockSpec((blk, D_IN), lambda i: (i, 0)),
          pl.BlockSpec((D_IN, 2 * C_OUT), lambda i: (0, 0)),
      ],
      out_specs=[
          pl.BlockSpec((blk, C_OUT), lambda i: (i, 0)),
          pl.BlockSpec((2, blk, C_HALF), lambda i: (0, i, 0)),
      ],
      out_shape=[
          jax.ShapeDtypeStruct((N_PAD, C_OUT), jnp.float32),
          jax.ShapeDtypeStruct((2, N_PAD, C_HALF), jnp.bfloat16),
      ],
  )(x_pad, wcat)


@functools.partial(
    pl.kernel,
    out_type=jax.ShapeDtypeStruct((2, N_PAD, C_HALF), jnp.float32),
    mesh=_mesh,
    scratch_types=[
        pltpu.VMEM((EROWS_PER_TILE + 8, EC), jnp.int32),
        pltpu.VMEM((EROWS_PER_TILE, EC), jnp.int32),
        pltpu.VMEM((EC, C_HALF), jnp.bfloat16),
        pltpu.VMEM((EC, C_HALF), jnp.bfloat16),
        pltpu.VMEM((EC, C_HALF), jnp.bfloat16),
        pltpu.VMEM((EC, C_HALF), jnp.bfloat16),
        pltpu.VMEM((EC, C_HALF), jnp.bfloat16),
        pltpu.VMEM((EC, C_HALF), jnp.bfloat16),
        pltpu.VMEM((EC, C_HALF), jnp.bfloat16),
        pltpu.VMEM((EC, C_HALF), jnp.bfloat16),
        pltpu.VMEM((EC, C_HALF), jnp.bfloat16),
        pltpu.VMEM((NPT, C_HALF), jnp.float32),
        pltpu.VMEM((NPT,), jnp.float32),
        pltpu.VMEM((NPT,), jnp.float32),
        pltpu.VMEM((NPT,), jnp.float32),
        pltpu.VMEM((NPT,), jnp.float32),
        pltpu.VMEM((NPT,), jnp.float32),
        pltpu.VMEM_SHARED((N_PAD, C_HALF), jnp.bfloat16),
        pltpu.VMEM_SHARED((N_PAD, C_HALF), jnp.bfloat16),
        pltpu.SemaphoreType.DMA,
        pltpu.SemaphoreType.DMA,
        pltpu.SemaphoreType.DMA,
        pltpu.SemaphoreType.DMA,
    ],
    compiler_params=pltpu.CompilerParams(
        use_tc_tiling_on_sc=False, needs_layout_passes=False),
)
def _sc_hops(yab_hbm, row_hbm, col_hbm, degp_hbm, s_out,
             rowb, colb, eb0, eb1, eb2, eb3, eb4, eb5, eb6, eb7, zbuf, sbuf,
             dp0, dp1, d1b, dnb, dsqb, u_sh, acc_sh,
             sem_g, sem_s, sem_w, sem_z):
  abuf, ubuf = eb2, eb3
  cid = lax.axis_index("c")
  sid = lax.axis_index("s")
  nbase = sid * NPT

  pltpu.sync_copy(row_hbm.at[pl.ds(sid * EROWS_PER_TILE, EROWS_PER_TILE)],
                  rowb.at[pl.ds(0, EROWS_PER_TILE)])
  def padrow(i, _):
    g = i % 8
    rowb[EROWS_PER_TILE + i // 8, pl.ds(g * 16, 16)] = (
        N_NODES + g * 16 + lax.iota(jnp.int32, 16))
    return 0
  lax.fori_loop(0, 8 * 8, padrow, 0)
  pltpu.sync_copy(col_hbm.at[pl.ds(sid * EROWS_PER_TILE, EROWS_PER_TILE)], colb)

  pltpu.sync_copy(degp_hbm.at[0, pl.ds(nbase, NPT)], dp0)
  pltpu.sync_copy(degp_hbm.at[1, pl.ds(nbase, NPT)], dp1)

  def dblk(t, _):
    sl = pl.ds(t * 16, 16)
    deg = dp0[sl] + dp1[sl] + 1.0
    i = plsc.bitcast(deg, jnp.int32)
    y = plsc.bitcast(jnp.int32(0x5F3759DF) - (i >> 1), jnp.float32)
    y = y * (1.5 - 0.5 * deg * y * y)
    y = y * (1.5 - 0.5 * deg * y * y)
    y = y * (1.5 - 0.5 * deg * y * y)
    dnb[sl] = y
    d1b[sl] = y * y
    dsqb[sl] = deg * y
    return 0
  lax.fori_loop(0, NPT // 16, dblk, 0)

  def zrow(i, _):
    zbuf[i, :] = jnp.zeros((C_HALF,), jnp.bfloat16)
    return 0
  lax.fori_loop(0, EC, zrow, 0)
  _fill2d(sbuf, NPT, C_HALF, 0.0, jnp.float32)

  def uchunk(j, _):
    base = nbase + j * EC
    pltpu.sync_copy(yab_hbm.at[cid, pl.ds(base, EC)], ubuf)

    def rblk(t, _):
      dvec = dnb[pl.ds(j * EC + t * 16, 16)]
      for rr in range(16):
        r = t * 16 + rr
        b0, b1 = plsc.unpack(ubuf[r, :], format=_INTER)
        ubuf[r, :] = plsc.pack(b0 * dvec[rr], b1 * dvec[rr], format=_INTER)
      return 0
    lax.fori_loop(0, EC // 16, rblk, 0)
    pltpu.sync_copy(ubuf, u_sh.at[pl.ds(base, EC)])
    pltpu.sync_copy(zbuf, acc_sh.at[pl.ds(base, EC)])
    return 0
  lax.fori_loop(0, NCH, uchunk, 0)
  plsc.subcore_barrier()

  def k_body(k, _):
    def wait_g(buf):
      pltpu.make_async_copy(u_sh.at[rowb.at[0]], buf, sem_g).wait()

    def wait_s(buf):
      pltpu.make_async_copy(buf, acc_sh.at[colb.at[0]], sem_s).wait()

    def gath(j, buf):
      pltpu.async_copy(u_sh.at[rowb.at[j]], buf, sem_g)

    def scat(j, buf):
      pltpu.async_copy(buf, acc_sh.at[colb.at[j]], sem_s, add=True)

    ebs = [eb0, eb1, eb2, eb3, eb4, eb5, eb6, eb7]
    for m in range(4):
      gath(m, ebs[m])
      pltpu.async_copy(zbuf, acc_sh.at[rowb.at[EROWS_PER_TILE]], sem_s,
                       add=True)

    def estep(jj, _):
      j0 = 8 * jj
      for m in range(8):
        wait_g(ebs[m])
        wait_s(ebs[(m + 4) % 8])
        gath(j0 + m + 4, ebs[(m + 4) % 8])
        scat(j0 + m, ebs[m])
      return 0
    lax.fori_loop(0, EROWS_PER_TILE // 8, estep, 0)
    for m in range(4):
      wait_g(ebs[m])
      wait_s(ebs[m])
    plsc.subcore_barrier()

    def nsl(j):
      return pl.ds(nbase + j * EC, EC)

    def rblk_fn(j, a, ub):
      def rblk(t, _):
        d1vec = d1b[pl.ds(j * EC + t * 16, 16)]
        for rr in range(16):
          r = t * 16 + rr
          dscale = d1vec[rr]
          a0, a1 = plsc.unpack(a[r, :], format=_INTER)
          b0, b1 = plsc.unpack(ub[r, :], format=_INTER)
          v0 = (a0 + b0) * dscale
          v1 = (a1 + b1) * dscale
          ub[r, :] = plsc.pack(v0, v1, format=_INTER)
          sl0 = pl.ds(0, 16)
          sl1 = pl.ds(16, 16)
          sbuf[j * EC + r, sl0] = sbuf[j * EC + r, sl0] + v0
          sbuf[j * EC + r, sl1] = sbuf[j * EC + r, sl1] + v1
        return 0
      lax.fori_loop(0, EC // 16, rblk, 0)

    bufs = [(eb0, eb1), (eb2, eb3)]
    pltpu.async_copy(acc_sh.at[nsl(0)], eb0, sem_g)
    pltpu.async_copy(u_sh.at[nsl(0)], eb1, sem_g)
    for j in range(NCH):
      a, ub = bufs[j % 2]
      pltpu.make_async_copy(acc_sh.at[nsl(0)], a, sem_g).wait()
      pltpu.make_async_copy(u_sh.at[nsl(0)], ub, sem_g).wait()
      if j >= 1:
        pltpu.make_async_copy(ub, u_sh.at[nsl(0)], sem_w).wait()
      if j + 1 < NCH:
        na, nb = bufs[(j + 1) % 2]
        pltpu.async_copy(acc_sh.at[nsl(j + 1)], na, sem_g)
        pltpu.async_copy(u_sh.at[nsl(j + 1)], nb, sem_g)
      pltpu.async_copy(zbuf, acc_sh.at[nsl(j)], sem_z)
      rblk_fn(j, a, ub)
      pltpu.async_copy(ub, u_sh.at[nsl(j)], sem_w)
    pltpu.make_async_copy(eb1, u_sh.at[nsl(0)], sem_w).wait()
    for j in range(NCH):
      pltpu.make_async_copy(zbuf, acc_sh.at[nsl(0)], sem_z).wait()
    plsc.subcore_barrier()
    return 0
  lax.fori_loop(0, K_HOPS, k_body, 0)

  def oblk(t, _):
    dvec = dsqb[pl.ds(t * 16, 16)]
    for rr in range(16):
      r = t * 16 + rr
      for g in range(2):
        sl = pl.ds(g * 16, 16)
        sbuf[r, sl] = sbuf[r, sl] * dvec[rr]
    return 0
  lax.fori_loop(0, NPT // 16, oblk, 0)
  pltpu.sync_copy(sbuf, s_out.at[cid, pl.ds(nbase, NPT)])


def _tc_final_body(y0_ref, s2_ref, b_ref, p_ref, o_ref):
  s_cat = jnp.concatenate([s2_ref[0], s2_ref[1]], axis=1)
  logits = ALPHA * y0_ref[...] + COEF * s_cat + b_ref[...]
  m = jnp.max(logits, axis=1, keepdims=True)
  ex = jnp.exp(logits - m)
  lse = jnp.log(jnp.sum(ex, axis=1, keepdims=True)) + m
  o_ref[...] = jnp.dot(logits - lse, p_ref[...],
                       preferred_element_type=jnp.float32,
                       precision=lax.Precision.HIGHEST)


def _tc_final(y0, s2, b2, pmat):
  blk = 512
  grid = (N_PAD // blk,)
  return pl.pallas_call(
      _tc_final_body,
      grid=grid,
      in_specs=[
          pl.BlockSpec((blk, C_OUT), lambda i: (i, 0)),
          pl.BlockSpec((2, blk, C_HALF), lambda i: (0, i, 0)),
          pl.BlockSpec((1, C_OUT), lambda i: (0, 0)),
          pl.BlockSpec((C_OUT, C_OUT), lambda i: (0, 0)),
      ],
      out_specs=pl.BlockSpec((blk, C_OUT), lambda i: (i, 0)),
      out_shape=jax.ShapeDtypeStruct((N_PAD, C_OUT), jnp.float32),
  )(y0, s2, b2, pmat)


def kernel(x, edge_index, W, b):
  row = edge_index[0]
  col = edge_index[1]
  pad = EROWS * EC - E_EDGES
  rowp = jnp.concatenate(
      [row, jnp.full((pad,), N_NODES, jnp.int32)]).reshape(EROWS, EC)
  colp = jnp.concatenate(
      [col, jnp.full((pad,), N_NODES, jnp.int32)]).reshape(EROWS, EC)
  x_pad = jnp.pad(x, ((0, N_PAD - N_NODES), (0, 0)))
  wcat = jnp.concatenate([W[:, _F], W], axis=1)

  y0, yab = _tc_prep(x_pad, wcat)
  degp = _sc_degree(colp)
  s2 = _sc_hops(yab, rowp, colp, degp)
  pmat = jnp.asarray(np.eye(C_OUT, dtype=np.float32)[:, _F].T)
  out = _tc_final(y0, s2, b[_F].reshape(1, C_OUT), pmat)
  return out[:N_NODES]
